# Initial kernel scaffold; baseline (speedup 1.0000x reference)
import functools

import jax
import jax.numpy as jnp
from jax import lax
from jax.experimental import pallas as pl
from jax.experimental.pallas import tpu as pltpu
from jax.experimental.pallas import tpu_sc as plsc


def _node_tables_body(x_ref, w_ref, p_ref, q_ref, *, n, n_pad, ns):
  xw = jnp.dot(x_ref[...], w_ref[...], preferred_element_type=jnp.float32)
  p_ref[:n] = xw[:, :ns]
  q_ref[:n] = xw[:, ns:]
  pad = jnp.zeros((n_pad - n, ns), jnp.float32)
  p_ref[n:] = pad
  q_ref[n:] = pad


def _edge_term_body(ea_ref, wc_ref, b1_ref, c_ref, *, de):
  acc = jnp.broadcast_to(b1_ref[...], c_ref.shape).astype(jnp.float32)
  for j in range(de):
    acc = acc + ea_ref[:, j:j + 1] * wc_ref[j:j + 1, :]
  c_ref[...] = acc


def _finish_body(s_ref, ct_ref, w2_ref, b2_ref, o_ref):
  s = s_ref[0] + s_ref[1]
  cnt = jnp.sum(ct_ref[...], axis=1, keepdims=True)
  agg = jnp.dot(s, w2_ref[...], preferred_element_type=jnp.float32)
  agg = agg + cnt * b2_ref[...]
  o_ref[...] = agg / jnp.maximum(cnt, 1.0)


def _sc_edge_body(p_hbm, q_hbm, c_hbm, src_hbm, dst_hbm, s_out, cnt_out,
                  sidx, didx, pg, qg, cg, msg, cnt_l, zb, shared_s, gsem,
                  *, n_pad, rows_per_worker, chunks, kj, ns):
  cid = lax.axis_index("c")
  sid = lax.axis_index("s")
  wid = cid * 16 + sid
  rpt = n_pad // 16
  zero16 = jnp.zeros((ns,), jnp.float32)
  ones16 = jnp.ones((ns,), jnp.float32)

  @pl.loop(0, rpt)
  def _(i):
    zb[i, :] = zero16
    cnt_l[pl.ds(i * 16, 16)] = zero16

  pltpu.sync_copy(zb, shared_s.at[pl.ds(sid * rpt, rpt)])
  plsc.subcore_barrier()

  row0 = wid * rows_per_worker

  @pl.loop(0, chunks)
  def _(c):
    r = row0 + c * kj
    pltpu.sync_copy(src_hbm.at[pl.ds(r, kj)], sidx)
    pltpu.sync_copy(dst_hbm.at[pl.ds(r, kj)], didx)
    pltpu.sync_copy(c_hbm.at[pl.ds(r * 128, kj * 128)], cg)
    descs = []
    for j in range(kj):
      descs.append(pltpu.async_copy(
          p_hbm.at[sidx.at[j]], pg.at[pl.ds(j * 128, 128)], gsem))
      descs.append(pltpu.async_copy(
          q_hbm.at[didx.at[j]], qg.at[pl.ds(j * 128, 128)], gsem))
    for dsc in descs:
      dsc.wait()

    @pl.loop(0, kj * 128, step=4)
    def _(e0):
      for k in range(4):
        ei = e0 + k
        msg[ei, :] = jnp.maximum(pg[ei, :] + qg[ei, :] + cg[ei, :], 0.0)

    for j in range(kj):
      @pl.loop(0, 128, step=16)
      def _(t, j=j):
        iv = didx[j, pl.ds(t, 16)]
        plsc.addupdate_scatter(cnt_l, [iv], ones16)

    for j in range(kj):
      pltpu.sync_copy(msg.at[pl.ds(j * 128, 128)],
                      shared_s.at[didx.at[j]], add=True)

  plsc.subcore_barrier()
  pltpu.sync_copy(shared_s.at[pl.ds(sid * rpt, rpt)],
                  s_out.at[cid].at[pl.ds(sid * rpt, rpt)])
  pltpu.sync_copy(cnt_l, cnt_out.at[wid])


def kernel(x, edge_index, edge_attr, W1, b1, W2, b2):
  n, d = x.shape
  e = edge_index.shape[1]
  de = edge_attr.shape[1]
  ns = W1.shape[1]

  nw = 32
  kj = 8
  chunk_e = kj * 128
  chunks = -(-e // (nw * chunk_e))
  e_pad = nw * chunks * chunk_e
  rows_per_worker = chunks * kj
  rtot = e_pad // 128
  n_pad = ((n + 15) // 16) * 16 + 16

  src = edge_index[0]
  dst = edge_index[1]
  pad_e = e_pad - e
  idx_pad = jnp.full((pad_e,), n, jnp.int32)
  src2 = jnp.concatenate([src, idx_pad]).reshape(rtot, 128)
  dst2 = jnp.concatenate([dst, idx_pad]).reshape(rtot, 128)
  ea_pad = jnp.concatenate(
      [edge_attr, jnp.zeros((pad_e, de), jnp.float32)], axis=0)

  w1ab = jnp.concatenate([W1[:d], W1[d:2 * d]], axis=1)
  w1c = W1[2 * d:]

  p_tab, q_tab = pl.pallas_call(
      functools.partial(_node_tables_body, n=n, n_pad=n_pad, ns=ns),
      out_shape=(
          jax.ShapeDtypeStruct((n_pad, ns), jnp.float32),
          jax.ShapeDtypeStruct((n_pad, ns), jnp.float32),
      ),
  )(x, w1ab)

  be = 8192
  c_tab = pl.pallas_call(
      functools.partial(_edge_term_body, de=de),
      grid=(e_pad // be,),
      in_specs=[
          pl.BlockSpec((be, de), lambda i: (i, 0)),
          pl.BlockSpec((de, ns), lambda i: (0, 0)),
          pl.BlockSpec((1, ns), lambda i: (0, 0)),
      ],
      out_specs=pl.BlockSpec((be, ns), lambda i: (i, 0)),
      out_shape=jax.ShapeDtypeStruct((e_pad, ns), jnp.float32),
  )(ea_pad, w1c, b1.reshape(1, ns))

  mesh = plsc.VectorSubcoreMesh(core_axis_name="c", subcore_axis_name="s")
  sc_fn = pl.kernel(
      functools.partial(_sc_edge_body, n_pad=n_pad,
                        rows_per_worker=rows_per_worker, chunks=chunks,
                        kj=kj, ns=ns),
      out_type=(
          jax.ShapeDtypeStruct((2, n_pad, ns), jnp.float32),
          jax.ShapeDtypeStruct((nw, n_pad), jnp.float32),
      ),
      mesh=mesh,
      scratch_types=[
          pltpu.VMEM((kj, 128), jnp.int32),
          pltpu.VMEM((kj, 128), jnp.int32),
          pltpu.VMEM((chunk_e, ns), jnp.float32),
          pltpu.VMEM((chunk_e, ns), jnp.float32),
          pltpu.VMEM((chunk_e, ns), jnp.float32),
          pltpu.VMEM((chunk_e, ns), jnp.float32),
          pltpu.VMEM((n_pad,), jnp.float32),
          pltpu.VMEM((n_pad // 16, ns), jnp.float32),
          pltpu.VMEM_SHARED((n_pad, ns), jnp.float32),
          pltpu.SemaphoreType.DMA,
      ],
  )
  s_parts, cnt_parts = sc_fn(p_tab, q_tab, c_tab, src2, dst2)

  out = pl.pallas_call(
      _finish_body,
      out_shape=jax.ShapeDtypeStruct((n_pad, d), jnp.float32),
  )(s_parts, cnt_parts.T, W2, b2.reshape(1, d))

  return out[:n]


# speedup vs baseline: 4.8496x; 4.8496x over previous
import functools

import jax
import jax.numpy as jnp
from jax import lax
from jax.experimental import pallas as pl
from jax.experimental.pallas import tpu as pltpu
from jax.experimental.pallas import tpu_sc as plsc


def _node_tables_body(x_ref, w_ref, p_ref, q_ref, *, n, n_pad, ns):
  xw = jnp.dot(x_ref[...], w_ref[...], preferred_element_type=jnp.float32)
  p_ref[:n] = xw[:, :ns]
  q_ref[:n] = xw[:, ns:]
  pad = jnp.zeros((n_pad - n, ns), jnp.float32)
  p_ref[n:] = pad
  q_ref[n:] = pad


def _edge_term_body(ea_ref, wc_ref, b1_ref, c_ref, *, de):
  acc = jnp.broadcast_to(b1_ref[...], c_ref.shape).astype(jnp.float32)
  for j in range(de):
    acc = acc + ea_ref[:, j:j + 1] * wc_ref[j:j + 1, :]
  c_ref[...] = acc


def _finish_body(s_ref, ct_ref, w2_ref, b2_ref, o_ref):
  s = s_ref[0] + s_ref[1]
  cnt = jnp.sum(ct_ref[...], axis=1, keepdims=True)
  agg = jnp.dot(s, w2_ref[...], preferred_element_type=jnp.float32)
  agg = agg + cnt * b2_ref[...]
  o_ref[...] = agg / jnp.maximum(cnt, 1.0)


def _sc_edge_body(p_hbm, q_hbm, c_hbm, src_hbm, dst_hbm, s_out, cnt_out,
                  sidx, didx, pg, qg, cg, msg, cnt_l, zb, shared_s, gsem,
                  *, n_pad, rows_per_worker, chunks, kj, ns):
  cid = lax.axis_index("c")
  sid = lax.axis_index("s")
  wid = cid * 16 + sid
  rpt = n_pad // 16
  zero16 = jnp.zeros((ns,), jnp.float32)
  ones16 = jnp.ones((ns,), jnp.float32)

  @pl.loop(0, rpt)
  def _(i):
    zb[i, :] = zero16
    cnt_l[i, :] = zero16

  pltpu.sync_copy(zb, shared_s.at[pl.ds(sid * rpt, rpt)])
  plsc.subcore_barrier()

  row0 = wid * rows_per_worker

  @pl.loop(0, chunks)
  def _(c):
    r = row0 + c * kj
    pltpu.sync_copy(src_hbm.at[pl.ds(r, kj)], sidx)
    pltpu.sync_copy(dst_hbm.at[pl.ds(r, kj)], didx)
    pltpu.sync_copy(c_hbm.at[pl.ds(r * 128, kj * 128)], cg)
    descs = []
    for j in range(kj):
      descs.append(pltpu.async_copy(
          p_hbm.at[sidx.at[j]], pg.at[pl.ds(j * 128, 128)], gsem))
      descs.append(pltpu.async_copy(
          q_hbm.at[didx.at[j]], qg.at[pl.ds(j * 128, 128)], gsem))
    for dsc in descs:
      dsc.wait()

    @pl.loop(0, kj * 128, step=4)
    def _(e0):
      for k in range(4):
        ei = e0 + k
        msg[ei, :] = jnp.maximum(pg[ei, :] + qg[ei, :] + cg[ei, :], 0.0)

    for j in range(kj):
      @pl.loop(0, 128, step=16)
      def _(t, j=j):
        iv = didx[j, pl.ds(t, 16)]
        plsc.addupdate_scatter(cnt_l, [iv >> 4, iv & 15], ones16)

    for j in range(kj):
      pltpu.sync_copy(msg.at[pl.ds(j * 128, 128)],
                      shared_s.at[didx.at[j]], add=True)

  plsc.subcore_barrier()
  pltpu.sync_copy(shared_s.at[pl.ds(sid * rpt, rpt)],
                  s_out.at[cid].at[pl.ds(sid * rpt, rpt)])
  pltpu.sync_copy(cnt_l, cnt_out.at[wid])


def kernel(x, edge_index, edge_attr, W1, b1, W2, b2):
  n, d = x.shape
  e = edge_index.shape[1]
  de = edge_attr.shape[1]
  ns = W1.shape[1]

  nw = 32
  kj = 8
  chunk_e = kj * 128
  chunks = -(-e // (nw * chunk_e))
  e_pad = nw * chunks * chunk_e
  rows_per_worker = chunks * kj
  rtot = e_pad // 128
  n_pad = ((n + 1 + 127) // 128) * 128

  src = edge_index[0]
  dst = edge_index[1]
  pad_e = e_pad - e
  idx_pad = jnp.full((pad_e,), n, jnp.int32)
  src2 = jnp.concatenate([src, idx_pad]).reshape(rtot, 128)
  dst2 = jnp.concatenate([dst, idx_pad]).reshape(rtot, 128)
  ea_pad = jnp.concatenate(
      [edge_attr, jnp.zeros((pad_e, de), jnp.float32)], axis=0)

  w1ab = jnp.concatenate([W1[:d], W1[d:2 * d]], axis=1)
  w1c = W1[2 * d:]

  p_tab, q_tab = pl.pallas_call(
      functools.partial(_node_tables_body, n=n, n_pad=n_pad, ns=ns),
      out_shape=(
          jax.ShapeDtypeStruct((n_pad, ns), jnp.float32),
          jax.ShapeDtypeStruct((n_pad, ns), jnp.float32),
      ),
  )(x, w1ab)

  be = 8192
  c_tab = pl.pallas_call(
      functools.partial(_edge_term_body, de=de),
      grid=(e_pad // be,),
      in_specs=[
          pl.BlockSpec((be, de), lambda i: (i, 0)),
          pl.BlockSpec((de, ns), lambda i: (0, 0)),
          pl.BlockSpec((1, ns), lambda i: (0, 0)),
      ],
      out_specs=pl.BlockSpec((be, ns), lambda i: (i, 0)),
      out_shape=jax.ShapeDtypeStruct((e_pad, ns), jnp.float32),
  )(ea_pad, w1c, b1.reshape(1, ns))

  mesh = plsc.VectorSubcoreMesh(core_axis_name="c", subcore_axis_name="s")
  sc_fn = pl.kernel(
      functools.partial(_sc_edge_body, n_pad=n_pad,
                        rows_per_worker=rows_per_worker, chunks=chunks,
                        kj=kj, ns=ns),
      out_type=(
          jax.ShapeDtypeStruct((2, n_pad, ns), jnp.float32),
          jax.ShapeDtypeStruct((nw, n_pad // 16, 16), jnp.float32),
      ),
      mesh=mesh,
      compiler_params=pltpu.CompilerParams(
          needs_layout_passes=False, use_tc_tiling_on_sc=False),
      scratch_types=[
          pltpu.VMEM((kj, 128), jnp.int32),
          pltpu.VMEM((kj, 128), jnp.int32),
          pltpu.VMEM((chunk_e, ns), jnp.float32),
          pltpu.VMEM((chunk_e, ns), jnp.float32),
          pltpu.VMEM((chunk_e, ns), jnp.float32),
          pltpu.VMEM((chunk_e, ns), jnp.float32),
          pltpu.VMEM((n_pad // 16, 16), jnp.float32),
          pltpu.VMEM((n_pad // 16, ns), jnp.float32),
          pltpu.VMEM_SHARED((n_pad, ns), jnp.float32),
          pltpu.SemaphoreType.DMA,
      ],
  )
  s_parts, cnt_parts = sc_fn(p_tab, q_tab, c_tab, src2, dst2)

  out = pl.pallas_call(
      _finish_body,
      out_shape=jax.ShapeDtypeStruct((n_pad, d), jnp.float32),
  )(s_parts, cnt_parts.reshape(nw, n_pad).T, W2, b2.reshape(1, d))

  return out[:n]


# speedup vs baseline: 5.9937x; 1.2359x over previous
import functools

import jax
import jax.numpy as jnp
from jax import lax
from jax.experimental import pallas as pl
from jax.experimental.pallas import tpu as pltpu
from jax.experimental.pallas import tpu_sc as plsc


def _node_tables_body(x_ref, w_ref, p_ref, q_ref, *, n, n_pad, ns):
  xw = jnp.dot(x_ref[...], w_ref[...], preferred_element_type=jnp.float32)
  p_ref[:n] = xw[:, :ns]
  q_ref[:n] = xw[:, ns:]
  pad = jnp.zeros((n_pad - n, ns), jnp.float32)
  p_ref[n:] = pad
  q_ref[n:] = pad


def _finish_body(s_ref, ct_ref, w2_ref, b2_ref, o_ref):
  s = s_ref[0] + s_ref[1]
  cnt = jnp.sum(ct_ref[...], axis=1, keepdims=True)
  agg = jnp.dot(s, w2_ref[...], preferred_element_type=jnp.float32)
  agg = agg + cnt * b2_ref[...]
  o_ref[...] = agg / jnp.maximum(cnt, 1.0)


def _sc_edge_body(p_hbm, q_hbm, ea_hbm, w1c_hbm, src_hbm, dst_hbm,
                  s_out, cnt_out,
                  sidx, didx, pg, qg, eg, msg, wcb, cnt_l, zb, shared_s, gsem,
                  *, n_pad, rows_per_worker, chunks, kj, ns, de):
  cid = lax.axis_index("c")
  sid = lax.axis_index("s")
  wid = cid * 16 + sid
  rpt = n_pad // 16
  zero16 = jnp.zeros((ns,), jnp.float32)
  ones16 = jnp.ones((ns,), jnp.float32)

  pltpu.sync_copy(w1c_hbm, wcb)

  @pl.loop(0, rpt)
  def _(i):
    zb[i, :] = zero16
    cnt_l[i, :] = zero16

  pltpu.sync_copy(zb, shared_s.at[pl.ds(sid * rpt, rpt)])
  plsc.subcore_barrier()

  wc = [wcb[j, :] for j in range(de + 1)]
  row0 = wid * rows_per_worker

  @pl.loop(0, chunks)
  def _(c):
    r = row0 + c * kj
    pltpu.sync_copy(src_hbm.at[pl.ds(r, kj)], sidx)
    pltpu.sync_copy(dst_hbm.at[pl.ds(r, kj)], didx)
    pltpu.sync_copy(ea_hbm.at[pl.ds(r * 32, kj * 32)], eg)
    descs = []
    for j in range(kj):
      descs.append(pltpu.async_copy(
          p_hbm.at[sidx.at[j]], pg.at[pl.ds(j * 128, 128)], gsem))
      descs.append(pltpu.async_copy(
          q_hbm.at[didx.at[j]], qg.at[pl.ds(j * 128, 128)], gsem))
    for dsc in descs:
      dsc.wait()

    @pl.loop(0, kj * 32)
    def _(g):
      ev = eg[g, :]
      for k in range(4):
        ei = g * 4 + k
        acc = pg[ei, :] + qg[ei, :] + wc[de]
        for j in range(de):
          acc = acc + ev[de * k + j] * wc[j]
        msg[ei, :] = jnp.maximum(acc, 0.0)

    for j in range(kj):
      @pl.loop(0, 128, step=16)
      def _(t, j=j):
        iv = didx[j, pl.ds(t, 16)]
        plsc.addupdate_scatter(cnt_l, [iv >> 4, iv & 15], ones16)

    for j in range(kj):
      pltpu.sync_copy(msg.at[pl.ds(j * 128, 128)],
                      shared_s.at[didx.at[j]], add=True)

  plsc.subcore_barrier()
  pltpu.sync_copy(shared_s.at[pl.ds(sid * rpt, rpt)],
                  s_out.at[cid].at[pl.ds(sid * rpt, rpt)])
  pltpu.sync_copy(cnt_l, cnt_out.at[wid])


def kernel(x, edge_index, edge_attr, W1, b1, W2, b2):
  n, d = x.shape
  e = edge_index.shape[1]
  de = edge_attr.shape[1]
  ns = W1.shape[1]

  nw = 32
  kj = 8
  chunk_e = kj * 128
  chunks = -(-e // (nw * chunk_e))
  e_pad = nw * chunks * chunk_e
  rows_per_worker = chunks * kj
  rtot = e_pad // 128
  n_pad = ((n + 1 + 127) // 128) * 128

  src = edge_index[0]
  dst = edge_index[1]
  pad_e = e_pad - e
  idx_pad = jnp.full((pad_e,), n, jnp.int32)
  src2 = jnp.concatenate([src, idx_pad]).reshape(rtot, 128)
  dst2 = jnp.concatenate([dst, idx_pad]).reshape(rtot, 128)
  ea_pad = jnp.concatenate(
      [edge_attr, jnp.zeros((pad_e, de), jnp.float32)], axis=0)

  w1ab = jnp.concatenate([W1[:d], W1[d:2 * d]], axis=1)
  w1c = W1[2 * d:]

  p_tab, q_tab = pl.pallas_call(
      functools.partial(_node_tables_body, n=n, n_pad=n_pad, ns=ns),
      out_shape=(
          jax.ShapeDtypeStruct((n_pad, ns), jnp.float32),
          jax.ShapeDtypeStruct((n_pad, ns), jnp.float32),
      ),
  )(x, w1ab)

  w1cb = jnp.concatenate([w1c, b1.reshape(1, ns)], axis=0)

  mesh = plsc.VectorSubcoreMesh(core_axis_name="c", subcore_axis_name="s")
  sc_fn = pl.kernel(
      functools.partial(_sc_edge_body, n_pad=n_pad,
                        rows_per_worker=rows_per_worker, chunks=chunks,
                        kj=kj, ns=ns, de=de),
      out_type=(
          jax.ShapeDtypeStruct((2, n_pad, ns), jnp.float32),
          jax.ShapeDtypeStruct((nw, n_pad // 16, 16), jnp.float32),
      ),
      mesh=mesh,
      compiler_params=pltpu.CompilerParams(
          needs_layout_passes=False, use_tc_tiling_on_sc=False),
      scratch_types=[
          pltpu.VMEM((kj, 128), jnp.int32),
          pltpu.VMEM((kj, 128), jnp.int32),
          pltpu.VMEM((chunk_e, ns), jnp.float32),
          pltpu.VMEM((chunk_e, ns), jnp.float32),
          pltpu.VMEM((chunk_e // 4, 16), jnp.float32),
          pltpu.VMEM((chunk_e, ns), jnp.float32),
          pltpu.VMEM((de + 1, ns), jnp.float32),
          pltpu.VMEM((n_pad // 16, 16), jnp.float32),
          pltpu.VMEM((n_pad // 16, ns), jnp.float32),
          pltpu.VMEM_SHARED((n_pad, ns), jnp.float32),
          pltpu.SemaphoreType.DMA,
      ],
  )
  ea_flat = ea_pad.reshape(e_pad // 4, 4 * de)
  s_parts, cnt_parts = sc_fn(p_tab, q_tab, ea_flat, w1cb, src2, dst2)

  out = pl.pallas_call(
      _finish_body,
      out_shape=jax.ShapeDtypeStruct((n_pad, d), jnp.float32),
  )(s_parts, cnt_parts.reshape(nw, n_pad).T, W2, b2.reshape(1, d))

  return out[:n]


# speedup vs baseline: 6.7917x; 1.1331x over previous
import functools

import jax
import jax.numpy as jnp
from jax import lax
from jax.experimental import pallas as pl
from jax.experimental.pallas import tpu as pltpu
from jax.experimental.pallas import tpu_sc as plsc


def _node_tables_body(x_ref, w_ref, p_ref, q_ref, *, n, n_pad, ns):
  xw = jnp.dot(x_ref[...], w_ref[...], preferred_element_type=jnp.float32)
  p_ref[:n] = xw[:, :ns]
  q_ref[:n] = xw[:, ns:]
  pad = jnp.zeros((n_pad - n, ns), jnp.float32)
  p_ref[n:] = pad
  q_ref[n:] = pad


def _finish_body(s_ref, ct_ref, w2_ref, b2_ref, o_ref):
  s = s_ref[0] + s_ref[1]
  cnt = jnp.sum(ct_ref[...], axis=1, keepdims=True)
  agg = jnp.dot(s, w2_ref[...], preferred_element_type=jnp.float32)
  agg = agg + cnt * b2_ref[...]
  o_ref[...] = agg / jnp.maximum(cnt, 1.0)


def _sc_edge_body(p_hbm, q_hbm, ea_hbm, w1c_hbm, src_hbm, dst_hbm,
                  s_out, cnt_out,
                  sidx, didx, pg, qg, eg, msg, wcb, cnt_l, shared_s,
                  isem0, isem1, esem0, esem1, gsem0, gsem1, ssem0, ssem1,
                  *, n_pad, rows_per_worker, chunks, kj, ns, de):
  cid = lax.axis_index("c")
  sid = lax.axis_index("s")
  wid = cid * 16 + sid
  rpt = n_pad // 16
  zero16 = jnp.zeros((ns,), jnp.float32)
  ones16 = jnp.ones((ns,), jnp.float32)
  isem = [isem0, isem1]
  esem = [esem0, esem1]
  gsem = [gsem0, gsem1]
  ssem = [ssem0, ssem1]
  row0 = wid * rows_per_worker

  idx_d = {}
  ea_d = {}
  g_d = {}
  s_d = {}

  def issue_idx(c):
    r = row0 + c * kj
    b, p = c % 4, c % 2
    idx_d[c] = [
        pltpu.async_copy(src_hbm.at[pl.ds(r, kj)], sidx.at[b], isem[p]),
        pltpu.async_copy(dst_hbm.at[pl.ds(r, kj)], didx.at[b], isem[p]),
    ]
    ea_d[c] = pltpu.async_copy(
        ea_hbm.at[pl.ds(r * 32, kj * 32)], eg.at[p], esem[p])

  def fire_gathers(c):
    b, p = c % 4, c % 2
    ds = []
    for j in range(kj):
      ds.append(pltpu.async_copy(
          p_hbm.at[sidx.at[b].at[j]],
          pg.at[p].at[pl.ds(j * 128, 128)], gsem[p]))
      ds.append(pltpu.async_copy(
          q_hbm.at[didx.at[b].at[j]],
          qg.at[p].at[pl.ds(j * 128, 128)], gsem[p]))
    g_d[c] = ds

  pltpu.sync_copy(w1c_hbm, wcb)
  issue_idx(0)
  issue_idx(1)
  for dsc in idx_d.pop(0):
    dsc.wait()
  fire_gathers(0)

  @pl.loop(0, rpt)
  def _(i):
    msg[0, i, :] = zero16
    cnt_l[i, :] = zero16

  pltpu.sync_copy(msg.at[0].at[pl.ds(0, rpt)],
                  shared_s.at[pl.ds(sid * rpt, rpt)])
  plsc.subcore_barrier()

  wc = [wcb[j, :] for j in range(de + 1)]

  for c in range(chunks):
    p = c % 2
    b = c % 4
    if c >= 2:
      for dsc in s_d.pop(c - 2):
        dsc.wait()
    if c + 1 < chunks:
      for dsc in idx_d.pop(c + 1):
        dsc.wait()
      fire_gathers(c + 1)
    for dsc in g_d.pop(c):
      dsc.wait()
    ea_d.pop(c).wait()

    @pl.loop(0, kj * 32)
    def _(g, p=p):
      ev = eg[p, g, :]
      for k in range(4):
        ei = g * 4 + k
        acc = pg[p, ei, :] + qg[p, ei, :] + wc[de]
        for j in range(de):
          acc = acc + ev[de * k + j] * wc[j]
        msg[p, ei, :] = jnp.maximum(acc, 0.0)

    for j in range(kj):
      @pl.loop(0, 128, step=16)
      def _(t, j=j, b=b):
        iv = didx[b, j, pl.ds(t, 16)]
        plsc.addupdate_scatter(cnt_l, [iv >> 4, iv & 15], ones16)

    s_d[c] = [
        pltpu.async_copy(msg.at[p].at[pl.ds(j * 128, 128)],
                         shared_s.at[didx.at[b].at[j]], ssem[p], add=True)
        for j in range(kj)
    ]
    if c + 2 < chunks:
      issue_idx(c + 2)

  for c in sorted(s_d):
    for dsc in s_d.pop(c):
      dsc.wait()

  plsc.subcore_barrier()
  pltpu.sync_copy(shared_s.at[pl.ds(sid * rpt, rpt)],
                  s_out.at[cid].at[pl.ds(sid * rpt, rpt)])
  pltpu.sync_copy(cnt_l, cnt_out.at[wid])


def kernel(x, edge_index, edge_attr, W1, b1, W2, b2):
  n, d = x.shape
  e = edge_index.shape[1]
  de = edge_attr.shape[1]
  ns = W1.shape[1]

  nw = 32
  kj = 4
  chunk_e = kj * 128
  chunks = -(-e // (nw * chunk_e))
  e_pad = nw * chunks * chunk_e
  rows_per_worker = chunks * kj
  rtot = e_pad // 128
  n_pad = ((n + 1 + 127) // 128) * 128

  src = edge_index[0]
  dst = edge_index[1]
  pad_e = e_pad - e
  idx_pad = jnp.full((pad_e,), n, jnp.int32)
  src2 = jnp.concatenate([src, idx_pad]).reshape(rtot, 128)
  dst2 = jnp.concatenate([dst, idx_pad]).reshape(rtot, 128)
  ea_pad = jnp.concatenate(
      [edge_attr, jnp.zeros((pad_e, de), jnp.float32)], axis=0)

  w1ab = jnp.concatenate([W1[:d], W1[d:2 * d]], axis=1)
  w1c = W1[2 * d:]

  p_tab, q_tab = pl.pallas_call(
      functools.partial(_node_tables_body, n=n, n_pad=n_pad, ns=ns),
      out_shape=(
          jax.ShapeDtypeStruct((n_pad, ns), jnp.float32),
          jax.ShapeDtypeStruct((n_pad, ns), jnp.float32),
      ),
  )(x, w1ab)

  w1cb = jnp.concatenate([w1c, b1.reshape(1, ns)], axis=0)

  mesh = plsc.VectorSubcoreMesh(core_axis_name="c", subcore_axis_name="s")
  sc_fn = pl.kernel(
      functools.partial(_sc_edge_body, n_pad=n_pad,
                        rows_per_worker=rows_per_worker, chunks=chunks,
                        kj=kj, ns=ns, de=de),
      out_type=(
          jax.ShapeDtypeStruct((2, n_pad, ns), jnp.float32),
          jax.ShapeDtypeStruct((nw, n_pad // 16, 16), jnp.float32),
      ),
      mesh=mesh,
      compiler_params=pltpu.CompilerParams(
          needs_layout_passes=False, use_tc_tiling_on_sc=False),
      scratch_types=[
          pltpu.VMEM((4, kj, 128), jnp.int32),
          pltpu.VMEM((4, kj, 128), jnp.int32),
          pltpu.VMEM((2, chunk_e, ns), jnp.float32),
          pltpu.VMEM((2, chunk_e, ns), jnp.float32),
          pltpu.VMEM((2, chunk_e // 4, 16), jnp.float32),
          pltpu.VMEM((2, chunk_e, ns), jnp.float32),
          pltpu.VMEM((de + 1, ns), jnp.float32),
          pltpu.VMEM((n_pad // 16, 16), jnp.float32),
          pltpu.VMEM_SHARED((n_pad, ns), jnp.float32),
          pltpu.SemaphoreType.DMA,
          pltpu.SemaphoreType.DMA,
          pltpu.SemaphoreType.DMA,
          pltpu.SemaphoreType.DMA,
          pltpu.SemaphoreType.DMA,
          pltpu.SemaphoreType.DMA,
          pltpu.SemaphoreType.DMA,
          pltpu.SemaphoreType.DMA,
      ],
  )
  ea_flat = ea_pad.reshape(e_pad // 4, 4 * de)
  s_parts, cnt_parts = sc_fn(p_tab, q_tab, ea_flat, w1cb, src2, dst2)

  out = pl.pallas_call(
      _finish_body,
      out_shape=jax.ShapeDtypeStruct((n_pad, d), jnp.float32),
  )(s_parts, cnt_parts.reshape(nw, n_pad).T, W2, b2.reshape(1, d))

  return out[:n]


# speedup vs baseline: 9.1830x; 1.3521x over previous
import functools

import jax
import jax.numpy as jnp
from jax import lax
from jax.experimental import pallas as pl
from jax.experimental.pallas import tpu as pltpu
from jax.experimental.pallas import tpu_sc as plsc


def _node_tables_body(x_ref, w_ref, p_ref, q_ref, *, n, n_pad, ns):
  xw = jnp.dot(x_ref[...], w_ref[...], preferred_element_type=jnp.float32)
  p_ref[:n] = xw[:, :ns]
  q_ref[:n] = xw[:, ns:]
  pad = jnp.zeros((n_pad - n, ns), jnp.float32)
  p_ref[n:] = pad
  q_ref[n:] = pad


def _finish_body(s_ref, ct_ref, w2_ref, b2_ref, o_ref):
  s = s_ref[0] + s_ref[1]
  cnt = jnp.sum(ct_ref[...], axis=1, keepdims=True)
  agg = jnp.dot(s, w2_ref[...], preferred_element_type=jnp.float32)
  agg = agg + cnt * b2_ref[...]
  o_ref[...] = agg / jnp.maximum(cnt, 1.0)


def _sc_edge_body(p_hbm, q_hbm, ea_hbm, w1c_hbm, ei_hbm,
                  s_out, cnt_out,
                  sidx, didx, pg, qg, eg, msg, wcb, cnt_l, zb, shared_s,
                  isem0, isem1, esem0, esem1, gsem0, gsem1, ssem0, ssem1,
                  *, n_pad, chunk, chunks, ns, de):
  cid = lax.axis_index("c")
  sid = lax.axis_index("s")
  wid = cid * 16 + sid
  rpt = n_pad // 16
  gb = chunk // 16
  zero16 = jnp.zeros((ns,), jnp.float32)
  ones16 = jnp.ones((ns,), jnp.float32)
  isem = [isem0, isem1]
  esem = [esem0, esem1]
  gsem = [gsem0, gsem1]
  ssem = [ssem0, ssem1]
  row0 = wid * (chunk * chunks)
  erow0 = wid * ((chunk * chunks) // 4)

  def issue_idx(c):
    b, p = c % 4, c % 2
    base = row0 + c * chunk
    pltpu.async_copy(ei_hbm.at[0].at[pl.ds(base, chunk)],
                     sidx.at[b], isem[p])
    pltpu.async_copy(ei_hbm.at[1].at[pl.ds(base, chunk)],
                     didx.at[b], isem[p])
    pltpu.async_copy(ea_hbm.at[pl.ds(erow0 + c * (chunk // 4), chunk // 4)],
                     eg.at[p], esem[p])

  def wait_idx(c):
    b, p = c % 4, c % 2
    pltpu.make_async_copy(ei_hbm.at[0].at[pl.ds(0, chunk)],
                          sidx.at[b], isem[p]).wait()
    pltpu.make_async_copy(ei_hbm.at[1].at[pl.ds(0, chunk)],
                          didx.at[b], isem[p]).wait()

  def fire_gathers(c):
    b, p = c % 4, c % 2

    @pl.loop(0, gb)
    def _(t):
      sv = sidx[b, pl.ds(t * 16, 16)]
      dv = didx[b, pl.ds(t * 16, 16)]
      pltpu.async_copy(p_hbm.at[sv], pg.at[p].at[pl.ds(t * 16, 16)], gsem[p])
      pltpu.async_copy(q_hbm.at[dv], qg.at[p].at[pl.ds(t * 16, 16)], gsem[p])

  pltpu.sync_copy(w1c_hbm, wcb)
  issue_idx(0)
  issue_idx(1)
  wait_idx(0)
  fire_gathers(0)

  @pl.loop(0, rpt)
  def _(i):
    zb[i, :] = zero16
    cnt_l[i, :] = zero16

  pltpu.sync_copy(zb, shared_s.at[pl.ds(sid * rpt, rpt)])
  plsc.subcore_barrier()

  wc = [wcb[j, :] for j in range(de + 1)]

  for c in range(chunks):
    p = c % 2
    b = c % 4
    if c >= 2:
      pltpu.make_async_copy(p_hbm.at[pl.ds(0, chunk)],
                            msg.at[p], ssem[p]).wait()
    if c + 1 < chunks:
      wait_idx(c + 1)
      fire_gathers(c + 1)
    pltpu.make_async_copy(p_hbm.at[pl.ds(0, chunk)], pg.at[p], gsem[p]).wait()
    pltpu.make_async_copy(q_hbm.at[pl.ds(0, chunk)], qg.at[p], gsem[p]).wait()
    pltpu.make_async_copy(ea_hbm.at[pl.ds(0, chunk // 4)],
                          eg.at[p], esem[p]).wait()

    @pl.loop(0, chunk // 4)
    def _(g, p=p):
      ev = eg[p, g, :]
      for k in range(4):
        ei = g * 4 + k
        acc = pg[p, ei, :] + qg[p, ei, :] + wc[de]
        for j in range(de):
          acc = acc + ev[de * k + j] * wc[j]
        msg[p, ei, :] = jnp.maximum(acc, 0.0)

    @pl.loop(0, gb)
    def _(t, p=p, b=b):
      iv = didx[b, pl.ds(t * 16, 16)]
      plsc.addupdate_scatter(cnt_l, [iv >> 4, iv & 15], ones16)
      pltpu.async_copy(msg.at[p].at[pl.ds(t * 16, 16)],
                       shared_s.at[iv], ssem[p], add=True)

    if c + 2 < chunks:
      issue_idx(c + 2)

  for c in (chunks - 2, chunks - 1):
    pltpu.make_async_copy(p_hbm.at[pl.ds(0, chunk)],
                          msg.at[c % 2], ssem[c % 2]).wait()

  plsc.subcore_barrier()
  pltpu.sync_copy(shared_s.at[pl.ds(sid * rpt, rpt)],
                  s_out.at[cid].at[pl.ds(sid * rpt, rpt)])
  pltpu.sync_copy(cnt_l, cnt_out.at[wid])


def kernel(x, edge_index, edge_attr, W1, b1, W2, b2):
  n, d = x.shape
  e = edge_index.shape[1]
  de = edge_attr.shape[1]
  ns = W1.shape[1]

  nw = 32
  chunk = 400
  chunks = e // (nw * chunk)
  n_pad = ((n + 1 + 127) // 128) * 128

  w1ab = jnp.concatenate([W1[:d], W1[d:2 * d]], axis=1)
  w1cb = jnp.concatenate([W1[2 * d:], b1.reshape(1, ns)], axis=0)

  p_tab, q_tab = pl.pallas_call(
      functools.partial(_node_tables_body, n=n, n_pad=n_pad, ns=ns),
      out_shape=(
          jax.ShapeDtypeStruct((n_pad, ns), jnp.float32),
          jax.ShapeDtypeStruct((n_pad, ns), jnp.float32),
      ),
  )(x, w1ab)

  mesh = plsc.VectorSubcoreMesh(core_axis_name="c", subcore_axis_name="s")
  sc_fn = pl.kernel(
      functools.partial(_sc_edge_body, n_pad=n_pad, chunk=chunk,
                        chunks=chunks, ns=ns, de=de),
      out_type=(
          jax.ShapeDtypeStruct((2, n_pad, ns), jnp.float32),
          jax.ShapeDtypeStruct((nw, n_pad // 16, 16), jnp.float32),
      ),
      mesh=mesh,
      compiler_params=pltpu.CompilerParams(
          needs_layout_passes=False, use_tc_tiling_on_sc=False),
      scratch_types=[
          pltpu.VMEM((4, chunk), jnp.int32),
          pltpu.VMEM((4, chunk), jnp.int32),
          pltpu.VMEM((2, chunk, ns), jnp.float32),
          pltpu.VMEM((2, chunk, ns), jnp.float32),
          pltpu.VMEM((2, chunk // 4, 16), jnp.float32),
          pltpu.VMEM((2, chunk, ns), jnp.float32),
          pltpu.VMEM((de + 1, ns), jnp.float32),
          pltpu.VMEM((n_pad // 16, 16), jnp.float32),
          pltpu.VMEM((n_pad // 16, ns), jnp.float32),
          pltpu.VMEM_SHARED((n_pad, ns), jnp.float32),
          pltpu.SemaphoreType.DMA,
          pltpu.SemaphoreType.DMA,
          pltpu.SemaphoreType.DMA,
          pltpu.SemaphoreType.DMA,
          pltpu.SemaphoreType.DMA,
          pltpu.SemaphoreType.DMA,
          pltpu.SemaphoreType.DMA,
          pltpu.SemaphoreType.DMA,
      ],
  )
  ea4 = edge_attr.reshape(e // 4, 4 * de)
  s_parts, cnt_parts = sc_fn(p_tab, q_tab, ea4, w1cb, edge_index)

  out = pl.pallas_call(
      _finish_body,
      out_shape=jax.ShapeDtypeStruct((n_pad, d), jnp.float32),
  )(s_parts, cnt_parts.reshape(nw, n_pad).T, W2, b2.reshape(1, d))

  return out[:n]
